# baseline (speedup 1.0000x reference)
import functools

import jax
import jax.numpy as jnp
from jax import lax
from jax.experimental import pallas as pl
from jax.experimental.pallas import tpu as pltpu
from jax.experimental.pallas import tpu_sc as plsc

RW = 128
L = 16


@functools.lru_cache(maxsize=None)
def _build(n_cols: int, n_rows: int, emb: int, nc: int, ns: int):
    nw = nc * ns
    assert n_rows == nw * RW and emb % L == 0

    mesh = plsc.VectorSubcoreMesh(core_axis_name="c", subcore_axis_name="s")

    @functools.partial(
        pl.kernel,
        out_type=jax.ShapeDtypeStruct((n_cols, emb, n_rows), jnp.float32),
        mesh=mesh,
        scratch_types=[
            pltpu.VMEM((n_cols, RW), jnp.int32),
            pltpu.VMEM((2, RW, emb), jnp.float32),
            pltpu.VMEM((2, emb, RW), jnp.float32),
            pltpu.SemaphoreType.DMA,
            pltpu.SemaphoreType.DMA,
            pltpu.SemaphoreType.DMA,
            pltpu.SemaphoreType.DMA,
        ],
        compiler_params=pltpu.CompilerParams(
            use_tc_tiling_on_sc=False, needs_layout_passes=False),
    )
    def emb_kernel(table_hbm, idxt_hbm, out_hbm, idx_v, buf, buft,
                   gsem0, gsem1, osem0, osem1):
        wid = lax.axis_index("s") * nc + lax.axis_index("c")
        r0 = wid * RW
        pltpu.sync_copy(idxt_hbm.at[:, pl.ds(r0, RW)], idx_v)
        gsem = (gsem0, gsem1)
        osem = (osem0, osem1)

        def fire_gather(c, p):
            pltpu.async_copy(table_hbm.at[idx_v.at[c]], buf.at[p], gsem[p])

        def wait_gather(p):
            pltpu.make_async_copy(
                table_hbm.at[idx_v.at[0]], buf.at[p], gsem[p]).wait()

        def fire_out(c, p):
            pltpu.async_copy(
                buft.at[p], out_hbm.at[c, :, pl.ds(r0, RW)], osem[p])

        def wait_out(p):
            pltpu.make_async_copy(
                buft.at[p], out_hbm.at[0, :, pl.ds(0, RW)], osem[p]).wait()

        rvecs = [jnp.arange(L, dtype=jnp.int32) + rb * L
                 for rb in range(RW // L)]

        def transpose(p):
            for e in range(emb):
                evec = jnp.full((L,), e, dtype=jnp.int32)
                for rb in range(RW // L):
                    v = plsc.load_gather(buf.at[p], [rvecs[rb], evec])
                    buft[p, e, pl.ds(rb * L, L)] = v

        fire_gather(0, 0)
        fire_gather(1, 1)
        for c in (0, 1):
            wait_gather(c)
            transpose(c)
            fire_out(c, c)
            fire_gather(c + 2, c)

        def pair_body(t, carry):
            for p in range(2):
                c = 2 + 2 * t + p
                wait_gather(p)
                wait_out(p)
                transpose(p)
                fire_out(c, p)
                fire_gather(c + 2, p)
            return carry

        lax.fori_loop(0, (n_cols - 4) // 2, pair_body, 0)

        for c, p in ((n_cols - 2, 0), (n_cols - 1, 1)):
            wait_gather(p)
            wait_out(p)
            transpose(p)
            fire_out(jnp.int32(c), p)
        wait_out(0)
        wait_out(1)

    return emb_kernel


def kernel(indices, table):
    n_rows, n_cols = indices.shape
    emb = table.shape[1]
    info = plsc.get_sparse_core_info()
    emb_kernel = _build(n_cols, n_rows, emb, info.num_cores, info.num_subcores)
    out2 = emb_kernel(table, indices.T)
    return jnp.transpose(out2, (2, 0, 1))


# speedup vs baseline: 1.8007x; 1.8007x over previous
import functools

import jax
import jax.numpy as jnp
from jax import lax
from jax.experimental import pallas as pl
from jax.experimental.pallas import tpu as pltpu
from jax.experimental.pallas import tpu_sc as plsc

CA, CB = 96, 104
K = 4


@functools.lru_cache(maxsize=None)
def _build(n_rows: int, n_cols: int, emb: int, nc: int, ns: int):
    nw = nc * ns
    assert n_rows % nw == 0 and n_cols == CA + CB
    rows_per_w = n_rows // nw
    nchunks = rows_per_w * 2
    assert nchunks % (2 * K) == 0
    ngroups = nchunks // K
    npairs = (ngroups - 2) // 2

    mesh = plsc.VectorSubcoreMesh(core_axis_name="c", subcore_axis_name="s")

    @functools.partial(
        pl.kernel,
        out_type=jax.ShapeDtypeStruct((n_rows, n_cols, emb), jnp.float32),
        mesh=mesh,
        scratch_types=[
            pltpu.VMEM((rows_per_w, n_cols), jnp.int32),
            pltpu.VMEM((2 * K, CB, emb), jnp.float32),
            pltpu.SemaphoreType.DMA,
            pltpu.SemaphoreType.DMA,
            pltpu.SemaphoreType.DMA,
            pltpu.SemaphoreType.DMA,
        ],
        compiler_params=pltpu.CompilerParams(use_tc_tiling_on_sc=False),
    )
    def emb_kernel(table_hbm, idx_hbm, out_hbm, idx_v, rows_v,
                   gsem0, gsem1, osem0, osem1):
        wid = lax.axis_index("s") * nc + lax.axis_index("c")
        row0 = wid * rows_per_w
        pltpu.sync_copy(idx_hbm.at[pl.ds(row0, rows_per_w)], idx_v)
        gsem = (gsem0, gsem1)
        osem = (osem0, osem1)

        def _cw(b):
            return (0, CA) if b % 2 == 0 else (CA, CB)

        def fire_gathers(g, h):
            for b in range(K):
                c0, cw = _cw(b)
                pltpu.async_copy(
                    table_hbm.at[idx_v.at[2 * g + b // 2, pl.ds(c0, cw)]],
                    rows_v.at[h * K + b, pl.ds(0, cw)], gsem[h])

        def wait_gathers(h):
            for b in range(K):
                c0, cw = _cw(b)
                pltpu.make_async_copy(
                    table_hbm.at[idx_v.at[0, pl.ds(0, cw)]],
                    rows_v.at[h * K + b, pl.ds(0, cw)], gsem[h]).wait()

        def fire_outs(g, h):
            for b in range(K):
                c0, cw = _cw(b)
                pltpu.async_copy(
                    rows_v.at[h * K + b, pl.ds(0, cw)],
                    out_hbm.at[row0 + 2 * g + b // 2, pl.ds(c0, cw)],
                    osem[h])

        def wait_outs(h):
            for b in range(K):
                c0, cw = _cw(b)
                pltpu.make_async_copy(
                    rows_v.at[h * K + b, pl.ds(0, cw)],
                    out_hbm.at[row0, pl.ds(c0, cw)], osem[h]).wait()

        fire_gathers(0, 0)
        wait_gathers(0)
        fire_outs(0, 0)
        fire_gathers(1, 1)

        def pair_body(t, carry):
            g1 = 2 * t + 1
            wait_gathers(1)
            fire_outs(g1, 1)
            wait_outs(0)
            fire_gathers(g1 + 1, 0)
            wait_gathers(0)
            fire_outs(g1 + 1, 0)
            wait_outs(1)
            fire_gathers(g1 + 2, 1)
            return carry

        lax.fori_loop(0, npairs, pair_body, 0)

        wait_gathers(1)
        fire_outs(ngroups - 1, 1)
        wait_outs(0)
        wait_outs(1)

    return emb_kernel


def kernel(indices, table):
    n_rows, n_cols = indices.shape
    emb = table.shape[1]
    info = plsc.get_sparse_core_info()
    emb_kernel = _build(n_rows, n_cols, emb, info.num_cores, info.num_subcores)
    return emb_kernel(table, indices)
